# baseline (speedup 1.0000x reference)
import jax
import jax.numpy as jnp
from jax import lax
from jax.experimental import pallas as pl
from jax.experimental.pallas import tpu as pltpu
from jax.experimental.pallas import tpu_sc as plsc

N = 320000
D = 128
H = 128
NUM_SEG = 10000

BN = 6400
W = 128
TBL = 10240
GRID = N // BN


def _pool_body(mins_ref, maxs_ref, ids_ref, x_ref, w1_ref, b1_ref, w2_ref,
               b2_ref, acc_ref, cnt_ref):
    i = pl.program_id(0)

    @pl.when(i == 0)
    def _init():
        acc_ref[...] = jnp.zeros_like(acc_ref)
        cnt_ref[...] = jnp.zeros_like(cnt_ref)

    h = jnp.dot(x_ref[...], w1_ref[...], preferred_element_type=jnp.float32)
    h = jnp.maximum(h + b1_ref[...], 0.0)
    h = jnp.dot(h, w2_ref[...], preferred_element_type=jnp.float32)
    h = jnp.maximum(h + b2_ref[...], 0.0)

    h_hi = h.astype(jnp.bfloat16).astype(jnp.float32)
    h_lo = h - h_hi

    ids = ids_ref[0]
    w0 = mins_ref[i] // W
    npass = maxs_ref[i] // W - w0 + 1

    def one_window(p, _):
        base = pl.multiple_of((w0 + p) * W, W)
        iota = lax.broadcasted_iota(jnp.int32, (W, BN), 0) + base
        onehot = (iota == ids).astype(jnp.float32)
        dims = (((1,), (0,)), ((), ()))
        part = (lax.dot_general(onehot, h_hi, dims,
                                preferred_element_type=jnp.float32)
                + lax.dot_general(onehot, h_lo, dims,
                                  preferred_element_type=jnp.float32))
        acc_ref[pl.ds(base, W), :] = acc_ref[pl.ds(base, W), :] + part
        cnt_ref[pl.ds(base, W), :] = (
            cnt_ref[pl.ds(base, W), :] + jnp.sum(onehot, axis=1, keepdims=True))
        return 0

    lax.fori_loop(0, npass, one_window, 0)


def _finish_body(acc_ref, cnt_ref, w3_ref, b3_ref, w_out_ref):
    pooled = acc_ref[...] / cnt_ref[...]
    y = jnp.dot(pooled, w3_ref[...], preferred_element_type=jnp.float32)
    w_out_ref[...] = jnp.maximum(y + b3_ref[...], 0.0)


def _segment_weights(x, batch3d, mins, maxs, W1, b1, W2, b2, W3, b3):
    full = lambda i: (0, 0)
    smem = pl.BlockSpec(memory_space=pltpu.SMEM)
    acc, cnt = pl.pallas_call(
        _pool_body,
        grid=(GRID,),
        in_specs=[
            smem,
            smem,
            pl.BlockSpec((1, 1, BN), lambda i: (i, 0, 0)),
            pl.BlockSpec((BN, D), lambda i: (i, 0)),
            pl.BlockSpec((D, H), full),
            pl.BlockSpec((1, H), full),
            pl.BlockSpec((H, H), full),
            pl.BlockSpec((1, H), full),
        ],
        out_specs=[
            pl.BlockSpec((TBL, H), full),
            pl.BlockSpec((TBL, 1), full),
        ],
        out_shape=[
            jax.ShapeDtypeStruct((TBL, H), jnp.float32),
            jax.ShapeDtypeStruct((TBL, 1), jnp.float32),
        ],
    )(mins, maxs, batch3d, x, W1, b1.reshape(1, H), W2, b2.reshape(1, H))
    return pl.pallas_call(
        _finish_body,
        out_shape=jax.ShapeDtypeStruct((TBL, 1), jnp.float32),
    )(acc, cnt, W3, b3.reshape(1, 1))


NC = 2
NS = 16
L = 16
NW = NC * NS
CHUNK = N // NW


def _sc_body(w_hbm, batch_hbm, out_hbm, wtbl_v, ids_v, out_v):
    wid = lax.axis_index("s") * NC + lax.axis_index("c")
    base = wid * CHUNK

    pltpu.sync_copy(w_hbm, wtbl_v)
    pltpu.sync_copy(batch_hbm.at[pl.ds(base, CHUNK)], ids_v)

    def gath(k, _):
        ids = ids_v[pl.ds(k * L, L)]
        out_v[pl.ds(k * L, L)] = plsc.load_gather(wtbl_v, [ids])
        return 0

    lax.fori_loop(0, CHUNK // L, gath, 0)

    pltpu.sync_copy(out_v, out_hbm.at[pl.ds(base, CHUNK)])


def _sc_gather(weights, batch):
    mesh = plsc.VectorSubcoreMesh(core_axis_name="c", subcore_axis_name="s")
    return pl.kernel(
        _sc_body,
        out_type=jax.ShapeDtypeStruct((N,), jnp.float32),
        mesh=mesh,
        compiler_params=pltpu.CompilerParams(needs_layout_passes=False),
        scratch_types=[
            pltpu.VMEM((TBL,), jnp.float32),
            pltpu.VMEM((CHUNK,), jnp.int32),
            pltpu.VMEM((CHUNK,), jnp.float32),
        ],
    )(weights, batch)


def kernel(x, batch, W1, b1, W2, b2, W3, b3):
    batch = batch.astype(jnp.int32)
    x = x.astype(jnp.float32)
    batch3d = batch.reshape(GRID, 1, BN)
    mins = batch[:: BN]
    maxs = batch[BN - 1 :: BN]
    weights = _segment_weights(x, batch3d, mins, maxs, W1, b1, W2, b2, W3, b3)
    return _sc_gather(weights.reshape(TBL), batch)


# speedup vs baseline: 1.2965x; 1.2965x over previous
import jax
import jax.numpy as jnp
from jax import lax
from jax.experimental import pallas as pl
from jax.experimental.pallas import tpu as pltpu
from jax.experimental.pallas import tpu_sc as plsc

N = 320000
D = 128
H = 128
NUM_SEG = 10000

BN = 4000
W = 128
TBL = 10240
GRID = N // BN


def _pool_body(mins_ref, maxs_ref, ids_ref, x_ref, w1_ref, b1_ref, w2_ref,
               b2_ref, acc_ref, cnt_ref):
    i = pl.program_id(0)

    @pl.when(i == 0)
    def _init():
        acc_ref[...] = jnp.zeros_like(acc_ref)
        cnt_ref[...] = jnp.zeros_like(cnt_ref)

    h = jnp.dot(x_ref[...], w1_ref[...], preferred_element_type=jnp.float32)
    h = jnp.maximum(h + b1_ref[...], 0.0)
    h = jnp.dot(h, w2_ref[...], preferred_element_type=jnp.float32)
    h = jnp.maximum(h + b2_ref[...], 0.0)

    h_hi = h.astype(jnp.bfloat16).astype(jnp.float32)
    h_lo = h - h_hi
    h_cat = jnp.concatenate([h_hi, h_lo], axis=1)

    ids = ids_ref[0]
    w0 = mins_ref[i] // W
    npass = maxs_ref[i] // W - w0 + 1

    def one_window(p, _):
        base = pl.multiple_of((w0 + p) * W, W)
        iota = lax.broadcasted_iota(jnp.int32, (W, BN), 0) + base
        onehot = (iota == ids).astype(jnp.float32)
        dims = (((1,), (0,)), ((), ()))
        both = lax.dot_general(onehot, h_cat, dims,
                               preferred_element_type=jnp.float32)
        part = both[:, :H] + both[:, H:]
        acc_ref[pl.ds(base, W), :] = acc_ref[pl.ds(base, W), :] + part
        cnt_ref[pl.ds(base, W), :] = (
            cnt_ref[pl.ds(base, W), :] + jnp.sum(onehot, axis=1, keepdims=True))
        return 0

    lax.fori_loop(0, npass, one_window, 0)


def _finish_body(acc_ref, cnt_ref, w3_ref, b3_ref, w_out_ref):
    pooled = acc_ref[...] / cnt_ref[...]
    y = jnp.dot(pooled, w3_ref[...], preferred_element_type=jnp.float32)
    w_out_ref[...] = jnp.maximum(y + b3_ref[...], 0.0)


def _segment_weights(x, batch3d, mins, maxs, W1, b1, W2, b2, W3, b3):
    full = lambda i: (0, 0)
    smem = pl.BlockSpec(memory_space=pltpu.SMEM)
    acc, cnt = pl.pallas_call(
        _pool_body,
        grid=(GRID,),
        in_specs=[
            smem,
            smem,
            pl.BlockSpec((1, 1, BN), lambda i: (i, 0, 0)),
            pl.BlockSpec((BN, D), lambda i: (i, 0)),
            pl.BlockSpec((D, H), full),
            pl.BlockSpec((1, H), full),
            pl.BlockSpec((H, H), full),
            pl.BlockSpec((1, H), full),
        ],
        out_specs=[
            pl.BlockSpec((TBL, H), full),
            pl.BlockSpec((TBL, 1), full),
        ],
        out_shape=[
            jax.ShapeDtypeStruct((TBL, H), jnp.float32),
            jax.ShapeDtypeStruct((TBL, 1), jnp.float32),
        ],
    )(mins, maxs, batch3d, x, W1, b1.reshape(1, H), W2, b2.reshape(1, H))
    return pl.pallas_call(
        _finish_body,
        out_shape=jax.ShapeDtypeStruct((TBL, 1), jnp.float32),
    )(acc, cnt, W3, b3.reshape(1, 1))


NC = 2
NS = 16
L = 16
NW = NC * NS
CHUNK = N // NW


def _sc_body(w_hbm, batch_hbm, out_hbm, wtbl_v, ids_v, out_v):
    wid = lax.axis_index("s") * NC + lax.axis_index("c")
    base = wid * CHUNK

    pltpu.sync_copy(w_hbm, wtbl_v)
    pltpu.sync_copy(batch_hbm.at[pl.ds(base, CHUNK)], ids_v)

    def gath(k, _):
        ids = ids_v[pl.ds(k * L, L)]
        out_v[pl.ds(k * L, L)] = plsc.load_gather(wtbl_v, [ids])
        return 0

    lax.fori_loop(0, CHUNK // L, gath, 0)

    pltpu.sync_copy(out_v, out_hbm.at[pl.ds(base, CHUNK)])


def _sc_gather(weights, batch):
    mesh = plsc.VectorSubcoreMesh(core_axis_name="c", subcore_axis_name="s")
    return pl.kernel(
        _sc_body,
        out_type=jax.ShapeDtypeStruct((N,), jnp.float32),
        mesh=mesh,
        compiler_params=pltpu.CompilerParams(needs_layout_passes=False),
        scratch_types=[
            pltpu.VMEM((TBL,), jnp.float32),
            pltpu.VMEM((CHUNK,), jnp.int32),
            pltpu.VMEM((CHUNK,), jnp.float32),
        ],
    )(weights, batch)


def kernel(x, batch, W1, b1, W2, b2, W3, b3):
    batch = batch.astype(jnp.int32)
    x = x.astype(jnp.float32)
    batch3d = batch.reshape(GRID, 1, BN)
    mins = batch[:: BN]
    maxs = batch[BN - 1 :: BN]
    weights = _segment_weights(x, batch3d, mins, maxs, W1, b1, W2, b2, W3, b3)
    return _sc_gather(weights.reshape(TBL), batch)
